# Initial kernel scaffold; baseline (speedup 1.0000x reference)
import jax
import jax.numpy as jnp
from jax.experimental import pallas as pl


def kernel(x):
    raise NotImplementedError("write your pallas kernel here")


# speedup vs baseline: 1173.2482x; 1173.2482x over previous
import functools

import jax
import jax.numpy as jnp
from jax import lax
from jax.experimental import pallas as pl
from jax.experimental.pallas import tpu as pltpu
from jax.experimental.pallas import tpu_sc as plsc

REGION_NUM = 16
L = 16
NCH = 384
NPIX = 50176
NW = 32
CPW = NCH // NW
NSLICE = NPIX // L

_ATOL = float(jnp.finfo(jnp.float32).eps) * 4
_RTOL = 1e-5


def _sc_body(x_hbm, out_hbm, buf, in_sem, out_sem):
    cid = lax.axis_index("c")
    sid = lax.axis_index("s")
    wid = sid * 2 + cid
    base = wid * CPW

    def channel_body(j, _):
        ch = base + j
        pltpu.async_copy(x_hbm.at[ch], buf, in_sem).wait()

        def p1(i, carry):
            mn, mx = carry
            v = buf[pl.ds(i * L, L)]
            return jnp.minimum(mn, v), jnp.maximum(mx, v)

        init_mn = jnp.full((L,), jnp.inf, jnp.float32)
        init_mx = jnp.full((L,), -jnp.inf, jnp.float32)
        mnv, mxv = lax.fori_loop(0, NSLICE, p1, (init_mn, init_mx))
        mn = mnv[0]
        mx = mxv[0]
        for k in range(1, L):
            mn = jnp.minimum(mn, mnv[k])
            mx = jnp.maximum(mx, mxv[k])

        rng = mx - mn
        degenerate = rng <= (_ATOL + _RTOL * jnp.abs(mx))
        rng_v = jnp.full((L,), 1.0, jnp.float32) * rng
        inv_raw = jnp.full((L,), jnp.float32(REGION_NUM)) / rng_v
        inv = jnp.where(rng > 0.0, inv_raw, jnp.zeros((L,), jnp.float32))
        delta = jnp.where(degenerate, 0.0, rng * jnp.float32(1.0 / REGION_NUM))
        c0 = mn + 0.5 * delta

        def p2(i, _):
            v = buf[pl.ds(i * L, L)]
            t = (v - mn) * inv
            idi = jnp.minimum(t.astype(jnp.int32), REGION_NUM - 1)
            q = c0 + delta * idi.astype(jnp.float32)
            buf[pl.ds(i * L, L)] = q
            return 0

        lax.fori_loop(0, NSLICE, p2, 0)
        pltpu.async_copy(buf, out_hbm.at[ch], out_sem).wait()
        return 0

    lax.fori_loop(0, CPW, channel_body, 0)


@jax.jit
def _quantize(flat):
    mesh = plsc.VectorSubcoreMesh(core_axis_name="c", subcore_axis_name="s")
    f = functools.partial(
        pl.kernel,
        mesh=mesh,
        out_type=jax.ShapeDtypeStruct((NCH, NPIX), jnp.float32),
        scratch_types=[
            pltpu.VMEM((NPIX,), jnp.float32),
            pltpu.SemaphoreType.DMA,
            pltpu.SemaphoreType.DMA,
        ],
    )(_sc_body)
    return f(flat)


def kernel(x):
    B, C, H, W = x.shape
    flat = x.reshape(B * C, H * W)
    q = _quantize(flat)
    return q.reshape(B, C, H, W)


# speedup vs baseline: 3304.4447x; 2.8165x over previous
import functools

import jax
import jax.numpy as jnp
from jax import lax
from jax.experimental import pallas as pl
from jax.experimental.pallas import tpu as pltpu
from jax.experimental.pallas import tpu_sc as plsc

REGION_NUM = 16
L = 16
NCH = 384
NPIX = 50176
NW = 32
CPW = NCH // NW
NSLICE = NPIX // L
U = 8
NITER = NSLICE // U

_ATOL = float(jnp.finfo(jnp.float32).eps) * 4
_RTOL = 1e-5


def _tree_minmax(vs):
    mns = list(vs)
    mxs = list(vs)
    while len(mns) > 1:
        mns = [jnp.minimum(mns[i], mns[i + 1]) for i in range(0, len(mns), 2)]
        mxs = [jnp.maximum(mxs[i], mxs[i + 1]) for i in range(0, len(mxs), 2)]
    return mns[0], mxs[0]


def _sc_body(x_hbm, out_hbm, buf, in_sem, out_sem):
    cid = lax.axis_index("c")
    sid = lax.axis_index("s")
    wid = sid * 2 + cid
    base = wid * CPW

    def in_copy(j, slot):
        return pltpu.make_async_copy(x_hbm.at[base + j], buf.at[slot], in_sem)

    def out_copy(j, slot):
        return pltpu.make_async_copy(buf.at[slot], out_hbm.at[base + j],
                                     out_sem)

    in_copy(0, 0).start()
    for j in range(CPW):
        slot = j % 2
        in_copy(j, slot).wait()
        if j + 1 < CPW:
            if j >= 1:
                out_copy(j - 1, 1 - slot).wait()
            in_copy(j + 1, 1 - slot).start()

        def p1(i, carry):
            mn, mx = carry
            b = i * (L * U)
            vs = [buf[slot, pl.ds(b + u * L, L)] for u in range(U)]
            tmn, tmx = _tree_minmax(vs)
            return jnp.minimum(mn, tmn), jnp.maximum(mx, tmx)

        init_mn = jnp.full((L,), jnp.inf, jnp.float32)
        init_mx = jnp.full((L,), -jnp.inf, jnp.float32)
        mnv, mxv = lax.fori_loop(0, NITER, p1, (init_mn, init_mx))
        mn = mnv[0]
        mx = mxv[0]
        for k in range(1, L):
            mn = jnp.minimum(mn, mnv[k])
            mx = jnp.maximum(mx, mxv[k])

        rng = mx - mn
        degenerate = rng <= (_ATOL + _RTOL * jnp.abs(mx))
        rng_v = jnp.full((L,), 1.0, jnp.float32) * rng
        inv_raw = jnp.full((L,), jnp.float32(REGION_NUM)) / rng_v
        inv = jnp.where(rng > 0.0, inv_raw, jnp.zeros((L,), jnp.float32))
        delta = jnp.where(degenerate, 0.0, rng * jnp.float32(1.0 / REGION_NUM))
        c0 = mn + 0.5 * delta
        nmn_inv = -mn * inv

        def p2(i, _):
            b = i * (L * U)
            for u in range(U):
                v = buf[slot, pl.ds(b + u * L, L)]
                t = v * inv + nmn_inv
                idi = jnp.minimum(t.astype(jnp.int32), REGION_NUM - 1)
                buf[slot, pl.ds(b + u * L, L)] = (
                    c0 + delta * idi.astype(jnp.float32))
            return 0

        lax.fori_loop(0, NITER, p2, 0)
        out_copy(j, slot).start()

    out_copy(CPW - 2, (CPW - 2) % 2).wait()
    out_copy(CPW - 1, (CPW - 1) % 2).wait()


@jax.jit
def _quantize(flat):
    mesh = plsc.VectorSubcoreMesh(core_axis_name="c", subcore_axis_name="s")
    f = functools.partial(
        pl.kernel,
        mesh=mesh,
        out_type=jax.ShapeDtypeStruct((NCH, NPIX), jnp.float32),
        scratch_types=[
            pltpu.VMEM((2, NPIX), jnp.float32),
            pltpu.SemaphoreType.DMA,
            pltpu.SemaphoreType.DMA,
        ],
    )(_sc_body)
    return f(flat)


def kernel(x):
    B, C, H, W = x.shape
    flat = x.reshape(B * C, H * W)
    q = _quantize(flat)
    return q.reshape(B, C, H, W)
